# Initial kernel scaffold; baseline (speedup 1.0000x reference)
import jax
import jax.numpy as jnp
from jax.experimental import pallas as pl


def kernel(sequence_output, attention, mention_starts, hts, W_h, b_h, W_t, b_t):
    raise NotImplementedError("write your pallas kernel here")


# speedup vs baseline: 2.6786x; 2.6786x over previous
import jax
import jax.numpy as jnp
from jax import lax
from jax.experimental import pallas as pl
from jax.experimental.pallas import tpu as pltpu

B, L, H, HEADS, NE, M = 4, 512, 768, 12, 42, 3
EMB = 768
R = NE * NE
AT = 7
TA = NE // AT
ROWS = TA * NE


def _k1_body(idx_ref, seq_ref, att_ref, wht_ref, wtt_ref, bh_ref, bt_ref,
             pn_ref, eh_ref, et_ref, p_scr):
    seq = seq_ref[0]
    idx = idx_ref[0]
    iota = lax.broadcasted_iota(jnp.int32, (NE, L), 1)
    oh = [(idx[:, m:m + 1] == iota).astype(jnp.float32) for m in range(M)]
    es = [jnp.dot(o, seq, preferred_element_type=jnp.float32) for o in oh]
    mx = jnp.maximum(jnp.maximum(es[0], es[1]), es[2])
    ee = mx + jnp.log(jnp.exp(es[0] - mx) + jnp.exp(es[1] - mx)
                      + jnp.exp(es[2] - mx))
    eh_ref[0] = (jnp.dot(ee, wht_ref[...], preferred_element_type=jnp.float32)
                 + bh_ref[...])
    et_ref[0] = (jnp.dot(ee, wtt_ref[...], preferred_element_type=jnp.float32)
                 + bt_ref[...])
    g = (oh[0] + oh[1] + oh[2]) * (1.0 / M)
    for h in range(HEADS):
        ea = jnp.dot(g, att_ref[0, h], preferred_element_type=jnp.float32)
        contrib = ea[:, None, :] * ea[None, :, :]
        if h == 0:
            p_scr[...] = contrib
        else:
            p_scr[...] += contrib
    p = jnp.maximum(p_scr[...] * (1.0 / HEADS), 0.0)
    s = jnp.sum(p, axis=-1, keepdims=True)
    pn_ref[0] = (p / (s + 1e-10)).reshape(R, L)


def _k2_body(pn_ref, seq_ref, eh_ref, et_ref, wcat_ref, outh_ref, outt_ref):
    rs = jnp.dot(pn_ref[0, 0], seq_ref[0],
                 preferred_element_type=jnp.float32)
    z = jnp.dot(rs, wcat_ref[...],
                preferred_element_type=jnp.float32)
    eht = eh_ref[0, 0]
    etf = et_ref[0]
    zh = z[:, :EMB].reshape(TA, NE, EMB)
    zt = z[:, EMB:].reshape(TA, NE, EMB)
    outh_ref[0, 0] = jnp.tanh(zh + eht[:, None, :]).reshape(ROWS, EMB)
    outt_ref[0, 0] = jnp.tanh(zt + etf[None, :, :]).reshape(ROWS, EMB)


def kernel(sequence_output, attention, mention_starts, hts, W_h, b_h, W_t, b_t):
    idx = (mention_starts + 1).astype(jnp.int32)
    wht, whb = W_h[:H], W_h[H:]
    wtt, wtb = W_t[:H], W_t[H:]
    wcat = jnp.concatenate([whb, wtb], axis=1)
    bh2 = b_h.reshape(1, EMB)
    bt2 = b_t.reshape(1, EMB)

    pn, eh, et = pl.pallas_call(
        _k1_body,
        grid=(B,),
        in_specs=[
            pl.BlockSpec((1, NE, M), lambda b: (b, 0, 0)),
            pl.BlockSpec((1, L, H), lambda b: (b, 0, 0)),
            pl.BlockSpec((1, HEADS, L, L), lambda b: (b, 0, 0, 0)),
            pl.BlockSpec((H, EMB), lambda b: (0, 0)),
            pl.BlockSpec((H, EMB), lambda b: (0, 0)),
            pl.BlockSpec((1, EMB), lambda b: (0, 0)),
            pl.BlockSpec((1, EMB), lambda b: (0, 0)),
        ],
        out_specs=[
            pl.BlockSpec((1, R, L), lambda b: (b, 0, 0)),
            pl.BlockSpec((1, NE, EMB), lambda b: (b, 0, 0)),
            pl.BlockSpec((1, NE, EMB), lambda b: (b, 0, 0)),
        ],
        out_shape=[
            jax.ShapeDtypeStruct((B, R, L), jnp.float32),
            jax.ShapeDtypeStruct((B, NE, EMB), jnp.float32),
            jax.ShapeDtypeStruct((B, NE, EMB), jnp.float32),
        ],
        scratch_shapes=[pltpu.VMEM((NE, NE, L), jnp.float32)],
    )(idx, sequence_output, attention, wht, wtt, bh2, bt2)

    pn4 = pn.reshape(B, AT, ROWS, L)
    eh4 = eh.reshape(B, AT, TA, EMB)

    outh, outt = pl.pallas_call(
        _k2_body,
        grid=(B, AT),
        in_specs=[
            pl.BlockSpec((1, 1, ROWS, L), lambda b, t: (b, t, 0, 0)),
            pl.BlockSpec((1, L, H), lambda b, t: (b, 0, 0)),
            pl.BlockSpec((1, 1, TA, EMB), lambda b, t: (b, t, 0, 0)),
            pl.BlockSpec((1, NE, EMB), lambda b, t: (b, 0, 0)),
            pl.BlockSpec((H, 2 * EMB), lambda b, t: (0, 0)),
        ],
        out_specs=[
            pl.BlockSpec((1, 1, ROWS, EMB), lambda b, t: (b, t, 0, 0)),
            pl.BlockSpec((1, 1, ROWS, EMB), lambda b, t: (b, t, 0, 0)),
        ],
        out_shape=[
            jax.ShapeDtypeStruct((B, AT, ROWS, EMB), jnp.float32),
            jax.ShapeDtypeStruct((B, AT, ROWS, EMB), jnp.float32),
        ],
    )(pn4, sequence_output, eh4, et, wcat)
    outh = outh.reshape(B, R, EMB)
    outt = outt.reshape(B, R, EMB)

    p = (hts[:, :, 0] * NE + hts[:, :, 1]).astype(jnp.int32)
    hs_out = jnp.take_along_axis(outh, p[:, :, None], axis=1)
    ts_out = jnp.take_along_axis(outt, p[:, :, None], axis=1)
    return (hs_out.reshape(B, NE, NE, EMB), ts_out.reshape(B, NE, NE, EMB))
